# Initial kernel scaffold; baseline (speedup 1.0000x reference)
import jax
import jax.numpy as jnp
from jax.experimental import pallas as pl


def kernel(x, codebook, output_scale, temperature_raw, gumbel):
    raise NotImplementedError("write your pallas kernel here")


# speedup vs baseline: 35.5027x; 35.5027x over previous
import jax
import jax.numpy as jnp
from jax.experimental import pallas as pl
from jax.experimental.pallas import tpu as pltpu

B, S, D = 4, 576, 64
K = 8192
TOPK = 8
TEMP_MIN = 0.1
N = B * S
TILE = 128
GRID = N // TILE


def _fused_body(x_ref, gum_ref, cb_ref, itau_ref, scale_ref,
                soft_ref, hard_ref, out_ref, ent_ref, cbn_ref):
    i = pl.program_id(0)

    @pl.when(i == 0)
    def _init():
        cb = cb_ref[...]
        n = jnp.maximum(jnp.sqrt(jnp.sum(cb * cb, axis=-1, keepdims=True)), 1e-12)
        cbn_ref[...] = cb / n
        ent_ref[...] = jnp.zeros((1, 1), jnp.float32)

    inv_tau = itau_ref[0, 0]
    scale = scale_ref[0, 0]

    x = x_ref[...]
    xden = jnp.maximum(jnp.sqrt(jnp.sum(x * x, axis=-1, keepdims=True)), 1e-12)
    xn = x / xden
    logits = jax.lax.dot_general(
        xn, cbn_ref[...], (((1,), (1,)), ((), ())),
        preferred_element_type=jnp.float32)
    y = (logits + gum_ref[...]) * inv_tau

    m = jnp.max(y, axis=-1, keepdims=True)
    e = jnp.exp(y - m)
    ssum = jnp.sum(e, axis=-1, keepdims=True)
    soft = e / ssum
    soft_ref[...] = soft
    ent_tile = jnp.sum(jnp.sum(soft * jnp.log(soft + 1e-8), axis=1, keepdims=True),
                       axis=0, keepdims=True)
    ent_ref[...] += -ent_tile

    col = jax.lax.broadcasted_iota(jnp.int32, (TILE, K), 1)
    yw = y
    hard = jnp.zeros((TILE, K), jnp.float32)
    for _ in range(TOPK):
        mx = jnp.max(yw, axis=-1, keepdims=True)
        cand = jnp.where(yw == mx, col, K)
        amin = jnp.min(cand, axis=-1, keepdims=True)
        pick = col == amin
        hard = jnp.where(pick, 1.0, hard)
        yw = jnp.where(pick, -jnp.inf, yw)
    hard_ref[...] = hard
    out_ref[...] = jax.lax.dot_general(
        hard, cb_ref[...], (((1,), (0,)), ((), ())),
        preferred_element_type=jnp.float32) * scale


def kernel(x, codebook, output_scale, temperature_raw, gumbel):
    tau = jnp.clip(temperature_raw, TEMP_MIN, None)
    inv_tau = (1.0 / tau).reshape(1, 1).astype(jnp.float32)
    scale = output_scale.reshape(1, 1).astype(jnp.float32)
    x2 = x.reshape(N, D)
    g2 = gumbel.reshape(N, K)

    soft, hard, out, ent = pl.pallas_call(
        _fused_body,
        grid=(GRID,),
        in_specs=[
            pl.BlockSpec((TILE, D), lambda i: (i, 0)),
            pl.BlockSpec((TILE, K), lambda i: (i, 0)),
            pl.BlockSpec((K, D), lambda i: (0, 0)),
            pl.BlockSpec((1, 1), lambda i: (0, 0), memory_space=pltpu.SMEM),
            pl.BlockSpec((1, 1), lambda i: (0, 0), memory_space=pltpu.SMEM),
        ],
        out_specs=[
            pl.BlockSpec((TILE, K), lambda i: (i, 0)),
            pl.BlockSpec((TILE, K), lambda i: (i, 0)),
            pl.BlockSpec((TILE, D), lambda i: (i, 0)),
            pl.BlockSpec((1, 1), lambda i: (0, 0)),
        ],
        out_shape=[
            jax.ShapeDtypeStruct((N, K), jnp.float32),
            jax.ShapeDtypeStruct((N, K), jnp.float32),
            jax.ShapeDtypeStruct((N, D), jnp.float32),
            jax.ShapeDtypeStruct((1, 1), jnp.float32),
        ],
        scratch_shapes=[pltpu.VMEM((K, D), jnp.float32)],
    )(x2, g2, codebook, inv_tau, scale)

    output = out.reshape(B, S, D)
    entropy = ent[0, 0] / N
    return (output, soft.reshape(B, S, K), hard.reshape(B, S, K), entropy)


# speedup vs baseline: 61.9814x; 1.7458x over previous
import jax
import jax.numpy as jnp
from jax.experimental import pallas as pl
from jax.experimental.pallas import tpu as pltpu

B, S, D = 4, 576, 64
K = 8192
TOPK = 8
TEMP_MIN = 0.1
N = B * S
TILE = 128
GRID = N // TILE


def _fused_body(x_ref, gum_ref, cb_ref, itau_ref, scale_ref,
                soft_ref, hard_ref, out_ref, ent_ref, cbn_ref):
    i = pl.program_id(0)

    @pl.when(i == 0)
    def _init():
        cb = cb_ref[...]
        n = jnp.maximum(jnp.sqrt(jnp.sum(cb * cb, axis=-1, keepdims=True)), 1e-12)
        cbn_ref[...] = cb / n
        ent_ref[...] = jnp.zeros((1, 1), jnp.float32)

    inv_tau = itau_ref[0, 0]
    scale = scale_ref[0, 0]

    x = x_ref[...]
    xden = jnp.maximum(jnp.sqrt(jnp.sum(x * x, axis=-1, keepdims=True)), 1e-12)
    xn = x / xden
    logits = jax.lax.dot_general(
        xn, cbn_ref[...], (((1,), (1,)), ((), ())),
        preferred_element_type=jnp.float32)
    y = (logits + gum_ref[...]) * inv_tau

    m = jnp.max(y, axis=-1, keepdims=True)
    e = jnp.exp(y - m)
    ssum = jnp.sum(e, axis=-1, keepdims=True)
    soft = e / ssum
    soft_ref[...] = soft
    ent_tile = jnp.sum(jnp.sum(soft * jnp.log(soft + 1e-8), axis=1, keepdims=True),
                       axis=0, keepdims=True)
    ent_ref[...] += -ent_tile

    neg = jnp.float32(-jnp.inf)
    t8 = m
    yw = y
    for _ in range(TOPK - 1):
        yw = jnp.where(yw < t8, yw, neg)
        t8 = jnp.max(yw, axis=-1, keepdims=True)
    hard = jnp.where(y >= t8, 1.0, 0.0)
    cnt = jnp.sum(hard, axis=-1, keepdims=True)
    hard_ref[...] = hard

    @pl.when(jnp.max(cnt) > 8.0)
    def _repair():
        col = jax.lax.broadcasted_iota(jnp.int32, (TILE, K), 1)
        yy = y
        hd = jnp.zeros((TILE, K), jnp.float32)
        for _ in range(TOPK):
            mx = jnp.max(yy, axis=-1, keepdims=True)
            cand = jnp.where(yy == mx, col, K)
            amin = jnp.min(cand, axis=-1, keepdims=True)
            pick = col == amin
            hd = jnp.where(pick, 1.0, hd)
            yy = jnp.where(pick, neg, yy)
        hard_ref[...] = hd

    out_ref[...] = jax.lax.dot_general(
        hard_ref[...], cb_ref[...], (((1,), (0,)), ((), ())),
        preferred_element_type=jnp.float32) * scale


def kernel(x, codebook, output_scale, temperature_raw, gumbel):
    tau = jnp.clip(temperature_raw, TEMP_MIN, None)
    inv_tau = (1.0 / tau).reshape(1, 1).astype(jnp.float32)
    scale = output_scale.reshape(1, 1).astype(jnp.float32)
    x2 = x.reshape(N, D)
    g2 = gumbel.reshape(N, K)

    soft, hard, out, ent = pl.pallas_call(
        _fused_body,
        grid=(GRID,),
        in_specs=[
            pl.BlockSpec((TILE, D), lambda i: (i, 0)),
            pl.BlockSpec((TILE, K), lambda i: (i, 0)),
            pl.BlockSpec((K, D), lambda i: (0, 0)),
            pl.BlockSpec((1, 1), lambda i: (0, 0), memory_space=pltpu.SMEM),
            pl.BlockSpec((1, 1), lambda i: (0, 0), memory_space=pltpu.SMEM),
        ],
        out_specs=[
            pl.BlockSpec((TILE, K), lambda i: (i, 0)),
            pl.BlockSpec((TILE, K), lambda i: (i, 0)),
            pl.BlockSpec((TILE, D), lambda i: (i, 0)),
            pl.BlockSpec((1, 1), lambda i: (0, 0)),
        ],
        out_shape=[
            jax.ShapeDtypeStruct((N, K), jnp.float32),
            jax.ShapeDtypeStruct((N, K), jnp.float32),
            jax.ShapeDtypeStruct((N, D), jnp.float32),
            jax.ShapeDtypeStruct((1, 1), jnp.float32),
        ],
        scratch_shapes=[pltpu.VMEM((K, D), jnp.float32)],
    )(x2, g2, codebook, inv_tau, scale)

    output = out.reshape(B, S, D)
    entropy = ent[0, 0] / N
    return (output, soft.reshape(B, S, K), hard.reshape(B, S, K), entropy)
